# Initial kernel scaffold; baseline (speedup 1.0000x reference)
import jax
import jax.numpy as jnp
from jax.experimental import pallas as pl


def kernel(x, edge_index, teamplate_node_mask, target_indices, edge_list, emb, conv_W_self, conv_W_msg, conv_W_hyper, conv_b, ln_scale, ln_bias, lin_W, lin_b, lin_ln_scale, lin_ln_bias, out_W, out_b):
    raise NotImplementedError("write your pallas kernel here")


# speedup vs baseline: 3.3038x; 3.3038x over previous
import functools

import jax
import jax.numpy as jnp
from jax import lax
from jax.experimental import pallas as pl
from jax.experimental.pallas import tpu as pltpu
from jax.experimental.pallas import tpu_sc as plsc

N = 10000
D = 128
E = 320000
H = 160000
T = 1024

NW = 32
NP = 10240
PAD_IDX = NP - 1
EPT = 10112
HPT = 5120
HPF = NW * HPT
EC = EPT // 128
BC = HPT // 64
SC1 = HPT // 128
RPT = NP // 16

_mesh = plsc.VectorSubcoreMesh(core_axis_name="c", subcore_axis_name="s",
                               num_cores=2, num_subcores=16)


def _r16(x):
    return x.astype(jnp.bfloat16).astype(jnp.float32)


def _make_gather(nrows_out, csz, nchunks):
    @functools.partial(
        pl.kernel,
        out_type=jax.ShapeDtypeStruct((nrows_out, D), jnp.float32),
        mesh=_mesh,
        scratch_types=[
            pltpu.VMEM((csz,), jnp.int32),
            pltpu.VMEM((csz, D), jnp.float32),
            pltpu.SemaphoreType.DMA,
        ],
    )
    def k(table_hbm, idx_hbm, out_hbm, idx_v, rows_v, sem):
        wid = lax.axis_index("s") * 2 + lax.axis_index("c")
        base = wid * (csz * nchunks)
        for j in range(nchunks):
            o = base + j * csz
            pltpu.sync_copy(idx_hbm.at[pl.ds(o, csz)], idx_v)
            pltpu.async_copy(table_hbm.at[idx_v], rows_v, sem).wait()
            pltpu.sync_copy(rows_v, out_hbm.at[pl.ds(o, csz)])

    return k


_gather_np = _make_gather(NP, 64, 5)
_gather_tgt = _make_gather(T, 32, 1)


@functools.partial(
    pl.kernel,
    out_type=jax.ShapeDtypeStruct((HPF, D), jnp.float32),
    mesh=_mesh,
    scratch_types=[
        pltpu.VMEM((64,), jnp.int32),
        pltpu.VMEM((64,), jnp.int32),
        pltpu.VMEM((64,), jnp.int32),
        pltpu.VMEM((64, D), jnp.float32),
        pltpu.VMEM((64, D), jnp.float32),
        pltpu.VMEM((64, D), jnp.float32),
        pltpu.SemaphoreType.DMA,
        pltpu.SemaphoreType.DMA,
        pltpu.SemaphoreType.DMA,
    ],
)
def _sc_b0(h_hbm, ea_hbm, eb_hbm, ec_hbm, s_hbm,
           ia, ib, ic, ra, rb, rc, sem, sem2, sem3):
    wid = lax.axis_index("s") * 2 + lax.axis_index("c")
    hbase = wid * HPT

    def body(j, carry):
        o = hbase + j * 64
        pltpu.sync_copy(ea_hbm.at[pl.ds(o, 64)], ia)
        pltpu.sync_copy(eb_hbm.at[pl.ds(o, 64)], ib)
        pltpu.sync_copy(ec_hbm.at[pl.ds(o, 64)], ic)
        da = pltpu.async_copy(h_hbm.at[ia], ra, sem)
        db = pltpu.async_copy(h_hbm.at[ib], rb, sem2)
        dc = pltpu.async_copy(h_hbm.at[ic], rc, sem3)
        da.wait()
        db.wait()
        dc.wait()

        def row_body(r, carry2):
            for q in range(8):
                sl = pl.ds(q * 16, 16)
                ra[r, sl] = ra[r, sl] + rb[r, sl] + rc[r, sl]
            return carry2

        lax.fori_loop(0, 64, row_body, 0)
        pltpu.sync_copy(ra, s_hbm.at[pl.ds(o, 64)])
        return carry

    lax.fori_loop(0, BC, body, 0)


@functools.partial(
    pl.kernel,
    out_type=jax.ShapeDtypeStruct((2, NP, D), jnp.float32),
    mesh=_mesh,
    scratch_types=[
        pltpu.VMEM_SHARED((NP, D), jnp.float32),
        pltpu.VMEM((128,), jnp.int32),
        pltpu.VMEM((128,), jnp.int32),
        pltpu.VMEM((128, D), jnp.float32),
        pltpu.SemaphoreType.DMA,
    ],
)
def _sc_a(mr_hbm, src_hbm, dst_hbm, zeros_hbm, accA_hbm,
          acc, i_s, i_d, rows, sem):
    c = lax.axis_index("c")
    s = lax.axis_index("s")
    wid = s * 2 + c
    my_rows = pl.ds(s * RPT, RPT)
    pltpu.sync_copy(zeros_hbm.at[my_rows], acc.at[my_rows])
    plsc.subcore_barrier()
    ebase = wid * EPT

    def body(j, carry):
        o = ebase + j * 128
        pltpu.sync_copy(src_hbm.at[pl.ds(o, 128)], i_s)
        pltpu.sync_copy(dst_hbm.at[pl.ds(o, 128)], i_d)
        pltpu.async_copy(mr_hbm.at[i_s], rows, sem).wait()
        pltpu.sync_copy(rows, acc.at[i_d], add=True)
        return carry

    lax.fori_loop(0, EC, body, 0)
    plsc.subcore_barrier()
    pltpu.sync_copy(acc.at[my_rows], accA_hbm.at[c, my_rows])


@functools.partial(
    pl.kernel,
    out_type=jax.ShapeDtypeStruct((2, NP, D), jnp.float32),
    mesh=_mesh,
    scratch_types=[
        pltpu.VMEM_SHARED((NP, D), jnp.float32),
        pltpu.VMEM((128,), jnp.int32),
        pltpu.VMEM((128,), jnp.int32),
        pltpu.VMEM((128,), jnp.int32),
        pltpu.VMEM((128, D), jnp.float32),
    ],
)
def _sc_b1(hef_hbm, ea_hbm, eb_hbm, ec_hbm, zeros_hbm, accB_hbm,
           acc, ia, ib, ic, buf):
    c = lax.axis_index("c")
    s = lax.axis_index("s")
    wid = s * 2 + c
    my_rows = pl.ds(s * RPT, RPT)
    pltpu.sync_copy(zeros_hbm.at[my_rows], acc.at[my_rows])
    plsc.subcore_barrier()
    hbase = wid * HPT

    def body(j, carry):
        o = hbase + j * 128
        pltpu.sync_copy(ea_hbm.at[pl.ds(o, 128)], ia)
        pltpu.sync_copy(eb_hbm.at[pl.ds(o, 128)], ib)
        pltpu.sync_copy(ec_hbm.at[pl.ds(o, 128)], ic)
        pltpu.sync_copy(hef_hbm.at[pl.ds(o, 128)], buf)
        pltpu.sync_copy(buf, acc.at[ia], add=True)
        pltpu.sync_copy(buf, acc.at[ib], add=True)
        pltpu.sync_copy(buf, acc.at[ic], add=True)
        return carry

    lax.fori_loop(0, SC1, body, 0)
    plsc.subcore_barrier()
    pltpu.sync_copy(acc.at[my_rows], accB_hbm.at[c, my_rows])


def _tc_mr_body(hr, wm, out_ref):
    p = jnp.dot(hr[...].astype(jnp.bfloat16), wm[...].astype(jnp.bfloat16),
                preferred_element_type=jnp.float32)
    out_ref[...] = _r16(p)


def _tc_mr(h_r, wm):
    R = 1280
    return pl.pallas_call(
        _tc_mr_body,
        grid=(NP // R,),
        in_specs=[pl.BlockSpec((R, D), lambda i: (i, 0)),
                  pl.BlockSpec((D, D), lambda i: (0, 0))],
        out_specs=pl.BlockSpec((R, D), lambda i: (i, 0)),
        out_shape=jax.ShapeDtypeStruct((NP, D), jnp.float32),
    )(h_r, wm)


def _tc_hef_body(s_ref, wh, out_ref):
    m = (s_ref[...] / 3.0).astype(jnp.bfloat16)
    p = jnp.dot(m, wh[...].astype(jnp.bfloat16),
                preferred_element_type=jnp.float32)
    out_ref[...] = _r16(p)


def _tc_hef(s, wh):
    R = 2560
    return pl.pallas_call(
        _tc_hef_body,
        grid=(HPF // R,),
        in_specs=[pl.BlockSpec((R, D), lambda i: (i, 0)),
                  pl.BlockSpec((D, D), lambda i: (0, 0))],
        out_specs=pl.BlockSpec((R, D), lambda i: (i, 0)),
        out_shape=jax.ShapeDtypeStruct((HPF, D), jnp.float32),
    )(s, wh)


def _tc_layer_body(hr, a0, a1, b0, b1, ws, bias, lns, lnb, out_ref, outr_ref):
    y = (jnp.dot(hr[...].astype(jnp.bfloat16), ws[...].astype(jnp.bfloat16),
                 preferred_element_type=jnp.float32)
         + (a0[...] + a1[...]) + (b0[...] + b1[...]) + bias[...])
    mu = jnp.mean(y, axis=1, keepdims=True)
    yc = y - mu
    var = jnp.mean(yc * yc, axis=1, keepdims=True)
    h = jnp.maximum(yc / jnp.sqrt(var + 1e-5) * lns[...] + lnb[...], 0.0)
    out_ref[...] = h
    outr_ref[...] = _r16(h)


def _tc_layer(h_r, a0, a1, b0, b1, ws, bias, lns, lnb):
    R = 1280
    row_spec = pl.BlockSpec((R, D), lambda i: (i, 0))
    w_spec = pl.BlockSpec((D, D), lambda i: (0, 0))
    v_spec = pl.BlockSpec((1, D), lambda i: (0, 0))
    return pl.pallas_call(
        _tc_layer_body,
        grid=(NP // R,),
        in_specs=[row_spec] * 5 + [w_spec] + [v_spec] * 3,
        out_specs=[row_spec, row_spec],
        out_shape=[jax.ShapeDtypeStruct((NP, D), jnp.float32),
                   jax.ShapeDtypeStruct((NP, D), jnp.float32)],
    )(h_r, a0, a1, b0, b1, ws, bias, lns, lnb)


def _tc_head_body(ht, w0, b0, s0, e0, w1, b1, s1, e1, ow, ob, out_ref):
    def ln_relu(v, sc, bi):
        mu = jnp.mean(v, axis=1, keepdims=True)
        vc = v - mu
        var = jnp.mean(vc * vc, axis=1, keepdims=True)
        return jnp.maximum(vc / jnp.sqrt(var + 1e-5) * sc + bi, 0.0)

    y = ln_relu(jnp.dot(ht[...].astype(jnp.bfloat16),
                        w0[...].astype(jnp.bfloat16),
                        preferred_element_type=jnp.float32) + b0[...],
                s0[...], e0[...])
    y = ln_relu(jnp.dot(y.astype(jnp.bfloat16), w1[...].astype(jnp.bfloat16),
                        preferred_element_type=jnp.float32) + b1[...],
                s1[...], e1[...])
    z = _r16(y) * _r16(ow[...])
    out_ref[...] = jnp.sum(z, axis=1, keepdims=True) + ob[...]


def _tc_head(ht, w0, b0, s0, e0, w1, b1, s1, e1, ow, ob):
    return pl.pallas_call(
        _tc_head_body,
        out_shape=jax.ShapeDtypeStruct((T, 1), jnp.float32),
    )(ht, w0, b0, s0, e0, w1, b1, s1, e1, ow, ob)


def kernel(x, edge_index, teamplate_node_mask, target_indices, edge_list,
           emb, conv_W_self, conv_W_msg, conv_W_hyper, conv_b,
           ln_scale, ln_bias, lin_W, lin_b, lin_ln_scale, lin_ln_bias,
           out_W, out_b):
    x_p = jnp.pad(jnp.ravel(x).astype(jnp.int32), (0, NP - N))
    src = jnp.pad(edge_index[0], (0, NW * EPT - E), constant_values=PAD_IDX)
    dst = jnp.pad(edge_index[1], (0, NW * EPT - E), constant_values=PAD_IDX)
    ea = jnp.pad(edge_list[:, 0], (0, HPF - H), constant_values=PAD_IDX)
    eb = jnp.pad(edge_list[:, 1], (0, HPF - H), constant_values=PAD_IDX)
    ec = jnp.pad(edge_list[:, 2], (0, HPF - H), constant_values=PAD_IDX)
    zeros = jnp.zeros((NP, D), jnp.float32)
    emb_r = _r16(emb)

    h = _gather_np(emb, x_p)
    h_r = _gather_np(emb_r, x_p)
    for i in range(2):
        s3 = _sc_b0(h, ea, eb, ec)
        mr = _tc_mr(h_r, conv_W_msg[i])
        accA = _sc_a(mr, src, dst, zeros)
        hef = _tc_hef(s3, conv_W_hyper[i])
        accB = _sc_b1(hef, ea, eb, ec, zeros)
        h, h_r = _tc_layer(h_r, accA[0], accA[1], accB[0], accB[1],
                           conv_W_self[i], conv_b[i][None, :],
                           ln_scale[i][None, :], ln_bias[i][None, :])
    ht_r = _gather_tgt(h_r, jnp.ravel(target_indices).astype(jnp.int32))
    return _tc_head(ht_r,
                    lin_W[0], lin_b[0][None, :], lin_ln_scale[0][None, :],
                    lin_ln_bias[0][None, :],
                    lin_W[1], lin_b[1][None, :], lin_ln_scale[1][None, :],
                    lin_ln_bias[1][None, :],
                    out_W.reshape(1, D), out_b.reshape(1, 1))
